# Initial kernel scaffold; baseline (speedup 1.0000x reference)
import functools

import jax
import jax.numpy as jnp
from jax import lax
from jax.experimental import pallas as pl
from jax.experimental.pallas import tpu as pltpu
from jax.experimental.pallas import tpu_sc as plsc

E = 320000
D = 128
NC, NS, L = 2, 16, 16
NW = NC * NS
EPW = E // NW
C = 400
NCHUNK = EPW // C
G = C // L

_mesh = plsc.VectorSubcoreMesh(core_axis_name="c", subcore_axis_name="s")


@functools.partial(
    pl.kernel,
    mesh=_mesh,
    out_type=jax.ShapeDtypeStruct((E,), jnp.float32),
    scratch_types=[
        pltpu.VMEM((C,), jnp.int32),
        pltpu.VMEM((C,), jnp.int32),
        pltpu.VMEM((C, D), jnp.float32),
        pltpu.VMEM((C, D), jnp.float32),
        pltpu.VMEM((C,), jnp.float32),
        pltpu.SemaphoreType.DMA,
        pltpu.SemaphoreType.DMA,
    ],
)
def _ipd(z_hbm, src_hbm, dst_hbm, out_hbm,
         si_v, di_v, srows_v, drows_v, out_v, sem_s, sem_d):
    wid = lax.axis_index("s") * NC + lax.axis_index("c")
    base = wid * EPW

    def chunk_body(c, carry):
        off = base + c * C
        pltpu.sync_copy(src_hbm.at[pl.ds(off, C)], si_v)
        pltpu.sync_copy(dst_hbm.at[pl.ds(off, C)], di_v)
        cp_s = pltpu.async_copy(z_hbm.at[si_v], srows_v, sem_s)
        cp_d = pltpu.async_copy(z_hbm.at[di_v], drows_v, sem_d)
        cp_s.wait()
        cp_d.wait()

        def group_body(g, gcarry):
            rows = g * L + lax.broadcasted_iota(jnp.int32, (L,), 0)

            def k_body(k, acc):
                ks = jnp.full((L,), k, jnp.int32)
                s = plsc.load_gather(srows_v, [rows, ks])
                d = plsc.load_gather(drows_v, [rows, ks])
                return acc + s * d

            acc = lax.fori_loop(0, D, k_body, jnp.zeros((L,), jnp.float32),
                                unroll=8)
            out_v[pl.ds(g * L, L)] = 1.0 / (1.0 + jnp.exp(-acc))
            return gcarry

        lax.fori_loop(0, G, group_body, 0)
        pltpu.sync_copy(out_v, out_hbm.at[pl.ds(off, C)])
        return carry

    lax.fori_loop(0, NCHUNK, chunk_body, 0)


def kernel(z, edge_index):
    ei = edge_index.astype(jnp.int32)
    return _ipd(z, ei[0], ei[1])


# speedup vs baseline: 1.1920x; 1.1920x over previous
import functools

import jax
import jax.numpy as jnp
from jax import lax
from jax.experimental import pallas as pl
from jax.experimental.pallas import tpu as pltpu
from jax.experimental.pallas import tpu_sc as plsc

E = 320000
D = 128
NC, NS, L = 2, 16, 16
NW = NC * NS
EPW = E // NW
C = 400
NCHUNK = EPW // C
G = C // L

_mesh = plsc.VectorSubcoreMesh(core_axis_name="c", subcore_axis_name="s")


@functools.partial(
    pl.kernel,
    mesh=_mesh,
    compiler_params=pltpu.CompilerParams(needs_layout_passes=False),
    out_type=jax.ShapeDtypeStruct((E,), jnp.float32),
    scratch_types=[
        pltpu.VMEM((C,), jnp.int32),
        pltpu.VMEM((C,), jnp.int32),
        pltpu.VMEM((C, D), jnp.float32),
        pltpu.VMEM((C, D), jnp.float32),
        pltpu.VMEM((C,), jnp.float32),
        pltpu.SemaphoreType.DMA,
        pltpu.SemaphoreType.DMA,
    ],
)
def _ipd(z_hbm, src_hbm, dst_hbm, out_hbm,
         si_v, di_v, srows_v, drows_v, out_v, sem_s, sem_d):
    wid = lax.axis_index("s") * NC + lax.axis_index("c")
    base = wid * EPW

    def chunk_body(c, carry):
        off = base + c * C
        pltpu.sync_copy(src_hbm.at[pl.ds(off, C)], si_v)
        pltpu.sync_copy(dst_hbm.at[pl.ds(off, C)], di_v)
        cp_s = pltpu.async_copy(z_hbm.at[si_v], srows_v, sem_s)
        cp_d = pltpu.async_copy(z_hbm.at[di_v], drows_v, sem_d)
        cp_s.wait()
        cp_d.wait()

        def group_body(g, gcarry):
            rows = g * L + lax.broadcasted_iota(jnp.int32, (L,), 0)

            def k_body(k, acc):
                ks = jnp.full((L,), k, jnp.int32)
                s = plsc.load_gather(srows_v, [rows, ks])
                d = plsc.load_gather(drows_v, [rows, ks])
                return acc + s * d

            acc = lax.fori_loop(0, D, k_body, jnp.zeros((L,), jnp.float32),
                                unroll=8)
            out_v[pl.ds(g * L, L)] = 1.0 / (1.0 + jnp.exp(-acc))
            return gcarry

        lax.fori_loop(0, G, group_body, 0)
        pltpu.sync_copy(out_v, out_hbm.at[pl.ds(off, C)])
        return carry

    lax.fori_loop(0, NCHUNK, chunk_body, 0)


def kernel(z, edge_index):
    ei = edge_index.astype(jnp.int32)
    return _ipd(z, ei[0], ei[1])


# speedup vs baseline: 5.7444x; 4.8193x over previous
import functools

import jax
import jax.numpy as jnp
from jax import lax
from jax.experimental import pallas as pl
from jax.experimental.pallas import tpu as pltpu
from jax.experimental.pallas import tpu_sc as plsc

E = 320000
D = 128
NC, NS, L = 2, 16, 16
NW = NC * NS
EPW = E // NW
C = 400
NCHUNK = EPW // C
G = C // L

_mesh = plsc.VectorSubcoreMesh(core_axis_name="c", subcore_axis_name="s")


@functools.partial(
    pl.kernel,
    mesh=_mesh,
    compiler_params=pltpu.CompilerParams(needs_layout_passes=False),
    out_type=jax.ShapeDtypeStruct((E,), jnp.float32),
    scratch_types=[
        pltpu.VMEM((C,), jnp.int32),
        pltpu.VMEM((C,), jnp.int32),
        pltpu.VMEM((C, D), jnp.float32),
        pltpu.VMEM((C, D), jnp.float32),
        pltpu.VMEM((C,), jnp.float32),
        pltpu.SemaphoreType.DMA,
        pltpu.SemaphoreType.DMA,
    ],
)
def _ipd(z_hbm, src_hbm, dst_hbm, out_hbm,
         si_v, di_v, srows_v, drows_v, out_v, sem_s, sem_d):
    wid = lax.axis_index("s") * NC + lax.axis_index("c")
    base = wid * EPW

    def chunk_body(c, carry):
        off = base + c * C
        pltpu.sync_copy(src_hbm.at[pl.ds(off, C)], si_v)
        pltpu.sync_copy(dst_hbm.at[pl.ds(off, C)], di_v)
        cp_s = pltpu.async_copy(z_hbm.at[si_v], srows_v, sem_s)
        cp_d = pltpu.async_copy(z_hbm.at[di_v], drows_v, sem_d)
        cp_s.wait()
        cp_d.wait()

        def group_body(g, gcarry):
            lane = lax.broadcasted_iota(jnp.int32, (L,), 0)
            rows = g * L + lane

            def k_body(k, acc):
                cols = (lane + k) & (D - 1)
                s = plsc.load_gather(srows_v, [rows, cols])
                d = plsc.load_gather(drows_v, [rows, cols])
                return acc + s * d

            acc = lax.fori_loop(0, D, k_body, jnp.zeros((L,), jnp.float32),
                                unroll=8)
            out_v[pl.ds(g * L, L)] = 1.0 / (1.0 + jnp.exp(-acc))
            return gcarry

        lax.fori_loop(0, G, group_body, 0)
        pltpu.sync_copy(out_v, out_hbm.at[pl.ds(off, C)])
        return carry

    lax.fori_loop(0, NCHUNK, chunk_body, 0)


def kernel(z, edge_index):
    ei = edge_index.astype(jnp.int32)
    return _ipd(z, ei[0], ei[1])


# speedup vs baseline: 8.8679x; 1.5437x over previous
import functools

import jax
import jax.numpy as jnp
from jax import lax
from jax.experimental import pallas as pl
from jax.experimental.pallas import tpu as pltpu
from jax.experimental.pallas import tpu_sc as plsc

E = 320000
D = 128
NC, NS, L = 2, 16, 16
NW = NC * NS
EPW = E // NW
CB = 80
NCHUNK = EPW // CB
GB = CB // L

_mesh = plsc.VectorSubcoreMesh(core_axis_name="c", subcore_axis_name="s")


@functools.partial(
    pl.kernel,
    mesh=_mesh,
    compiler_params=pltpu.CompilerParams(needs_layout_passes=False),
    out_type=jax.ShapeDtypeStruct((E,), jnp.float32),
    scratch_types=[
        pltpu.VMEM((EPW,), jnp.int32),
        pltpu.VMEM((EPW,), jnp.int32),
        pltpu.VMEM((CB, D), jnp.float32),
        pltpu.VMEM((CB, D), jnp.float32),
        pltpu.VMEM((CB, D), jnp.float32),
        pltpu.VMEM((CB, D), jnp.float32),
        pltpu.VMEM((EPW,), jnp.float32),
        pltpu.SemaphoreType.DMA,
        pltpu.SemaphoreType.DMA,
    ],
)
def _ipd(z_hbm, src_hbm, dst_hbm, out_hbm,
         si_v, di_v, sr0, sr1, dr0, dr1, out_v, sem0, sem1):
    wid = lax.axis_index("s") * NC + lax.axis_index("c")
    base = wid * EPW
    sbufs, dbufs, sems = (sr0, sr1), (dr0, dr1), (sem0, sem1)

    pltpu.sync_copy(src_hbm.at[pl.ds(base, EPW)], si_v)
    pltpu.sync_copy(dst_hbm.at[pl.ds(base, EPW)], di_v)

    def start(b, c):
        pltpu.async_copy(z_hbm.at[si_v.at[pl.ds(c * CB, CB)]], sbufs[b], sems[b])
        pltpu.async_copy(z_hbm.at[di_v.at[pl.ds(c * CB, CB)]], dbufs[b], sems[b])

    def drain(b):
        dummy = z_hbm.at[pl.ds(0, CB)]
        pltpu.make_async_copy(dummy, sbufs[b], sems[b]).wait()
        pltpu.make_async_copy(dummy, dbufs[b], sems[b]).wait()

    def compute(b, c):
        srows_v, drows_v = sbufs[b], dbufs[b]

        def group_body(g, gcarry):
            lane = lax.broadcasted_iota(jnp.int32, (L,), 0)
            rows = g * L + lane

            def k_body(k, acc):
                cols = (lane + k) & (D - 1)
                s = plsc.load_gather(srows_v, [rows, cols])
                d = plsc.load_gather(drows_v, [rows, cols])
                return acc + s * d

            acc = lax.fori_loop(0, D, k_body, jnp.zeros((L,), jnp.float32),
                                unroll=8)
            out_v[pl.ds(c * CB + g * L, L)] = 1.0 / (1.0 + jnp.exp(-acc))
            return gcarry

        lax.fori_loop(0, GB, group_body, 0)

    start(0, 0)
    start(1, 1)

    def chunk_body(c, carry):
        for b in (0, 1):
            @pl.when(c % 2 == b)
            def _():
                drain(b)
                compute(b, c)

                @pl.when(c + 2 < NCHUNK)
                def _():
                    start(b, c + 2)

        return carry

    lax.fori_loop(0, NCHUNK, chunk_body, 0)
    pltpu.sync_copy(out_v, out_hbm.at[pl.ds(base, EPW)])


def kernel(z, edge_index):
    ei = edge_index.astype(jnp.int32)
    return _ipd(z, ei[0], ei[1])
